# Initial kernel scaffold; baseline (speedup 1.0000x reference)
import jax
import jax.numpy as jnp
from jax.experimental import pallas as pl


def kernel(x, Wf1, bf1, Wf2, bf2, Wq, bq, Wk, bk):
    raise NotImplementedError("write your pallas kernel here")


# speedup vs baseline: 14.8535x; 14.8535x over previous
import functools

import jax
import jax.numpy as jnp
from jax import lax
from jax.experimental import pallas as pl
from jax.experimental.pallas import tpu as pltpu
from jax.experimental.pallas import tpu_sc as plsc

H = 64
W = 64
L = H * W
CIN = 64
QCH = 16
TOPK5 = 5
LB_B = 256
LB_D = 512
GW = 640
NEG = -3.0e38


def _unfold_perm(z):
    c = z.shape[0]
    zp = jnp.pad(z, ((0, 0), (1, 1), (1, 1)))
    cols = [zp[:, i:i + H, j:j + W].reshape(c, L)
            for i in range(3) for j in range(3)]
    return jnp.concatenate(cols, axis=0).T


def _wflat(w):
    o, i = w.shape[0], w.shape[1]
    return w.transpose(2, 3, 1, 0).reshape(9 * i, o)


def _stage_a_body(xu_ref, w1_ref, b1_ref, wq_ref, bq_ref, wk_ref, bk_ref,
                  h1_ref, q_ref, k_ref):
    xu = xu_ref[...]
    h1 = jnp.dot(xu, w1_ref[...], preferred_element_type=jnp.float32)
    h1_ref[...] = jnp.maximum(h1 + b1_ref[...], 0.0)
    q_ref[...] = jnp.dot(xu, wq_ref[...],
                         preferred_element_type=jnp.float32) + bq_ref[...]
    k_ref[...] = jnp.dot(xu, wk_ref[...],
                         preferred_element_type=jnp.float32) + bk_ref[...]


def _stage_b_body(h1u_ref, w2_ref, b2_ref, qu_ref, ku_ref,
                  feat_ref, vals_ref, idxs_ref):
    h1u = h1u_ref[...]
    f = jnp.dot(h1u, w2_ref[...], preferred_element_type=jnp.float32)
    feat_ref[...] = jnp.maximum(f + b2_ref[...], 0.0)

    qu = qu_ref[...]
    qn = qu / jnp.maximum(
        jnp.sqrt(jnp.sum(qu * qu, axis=1, keepdims=True)), 1e-12)
    ku = ku_ref[...]
    kn = ku / jnp.maximum(
        jnp.sqrt(jnp.sum(ku * ku, axis=1, keepdims=True)), 1e-12)
    r = lax.dot_general(qn, kn, (((1,), (1,)), ((), ())),
                        preferred_element_type=jnp.float32)

    iota = lax.broadcasted_iota(jnp.int32, (LB_B, L), 1)
    vs, ids = [], []
    for _ in range(TOPK5):
        m = jnp.max(r, axis=1, keepdims=True)
        idx = jnp.min(jnp.where(r == m, iota, L), axis=1, keepdims=True)
        vs.append(m)
        ids.append(idx)
        r = jnp.where(iota == idx, NEG, r)
    vals_ref[...] = jnp.concatenate(
        vs + [jnp.zeros((LB_B, 3), jnp.float32)], axis=1)
    idxs_ref[...] = jnp.concatenate(
        ids + [jnp.zeros((LB_B, 3), jnp.int32)], axis=1)


def _stage_d_body(gp_ref, gc_ref, gn_ref, vals_ref, x_ref, feat_ref, out_ref):
    pid = pl.program_id(0)
    l_loc = lax.broadcasted_iota(jnp.int32, (LB_D, 1), 0)
    l_glob = l_loc + pid * LB_D
    yy = l_glob // W
    xx = l_glob % W

    cat = jnp.concatenate([gp_ref[...], gc_ref[...], gn_ref[...]],
                          axis=1)
    total = jnp.zeros((LB_D, CIN), jnp.float32)
    for t in range(4):
        s = vals_ref[:, t + 1:t + 2]
        contrib = jnp.zeros((LB_D, CIN), jnp.float32)
        for di in (-1, 0, 1):
            for dj in (-1, 0, 1):
                kap = (di + 1) * 3 + (dj + 1)
                start = LB_D - 64 * di - dj
                sh = cat[t, start:start + LB_D, kap * CIN:(kap + 1) * CIN]
                valid = ((yy - di) >= 0) & ((yy - di) < H) & \
                        ((xx - dj) >= 0) & ((xx - dj) < W)
                contrib = contrib + jnp.where(valid, sh, 0.0)
        total = total + s * contrib
    out_ref[...] = x_ref[...] + feat_ref[...] + total * (1.0 / 36.0)


def _gather_rows(table, idx_flat):
    info = plsc.get_sparse_core_info()
    nw = info.num_cores * info.num_subcores
    n = idx_flat.shape[0]
    d = table.shape[1]
    per_w = n // nw
    ch = 128
    mesh = plsc.VectorSubcoreMesh(core_axis_name="c", subcore_axis_name="s")

    @functools.partial(
        pl.kernel, mesh=mesh,
        out_type=jax.ShapeDtypeStruct((n, d), jnp.float32),
        scratch_types=[
            pltpu.VMEM((ch,), jnp.int32),
            pltpu.VMEM((ch, d), jnp.float32),
            pltpu.SemaphoreType.DMA,
        ],
    )
    def k(table_hbm, idx_hbm, out_hbm, idx_v, rows_v, sem):
        wid = lax.axis_index("s") * info.num_cores + lax.axis_index("c")
        base = wid * per_w
        for c in range(per_w // ch):
            off = base + c * ch
            pltpu.sync_copy(idx_hbm.at[pl.ds(off, ch)], idx_v)
            pltpu.async_copy(table_hbm.at[idx_v], rows_v, sem).wait()
            pltpu.sync_copy(rows_v, out_hbm.at[pl.ds(off, ch)])

    return k(table, idx_flat)


def kernel(x, Wf1, bf1, Wf2, bf2, Wq, bq, Wk, bk):
    x_sp = x.reshape(CIN, H, W)
    xu = _unfold_perm(x_sp)
    xu_p = jnp.pad(xu, ((0, 0), (0, GW - 9 * CIN)))
    w1 = jnp.pad(_wflat(Wf1), ((0, GW - 9 * CIN), (0, 0)))
    w2 = _wflat(Wf2)
    wq = jnp.pad(_wflat(Wq), ((0, GW - 9 * CIN), (0, 0)))
    wk = jnp.pad(_wflat(Wk), ((0, GW - 9 * CIN), (0, 0)))
    b1 = bf1.reshape(1, CIN)
    b2 = bf2.reshape(1, CIN)
    bqr = bq.reshape(1, QCH)
    bkr = bk.reshape(1, QCH)

    h1, q, k = pl.pallas_call(
        _stage_a_body,
        out_shape=[
            jax.ShapeDtypeStruct((L, CIN), jnp.float32),
            jax.ShapeDtypeStruct((L, QCH), jnp.float32),
            jax.ShapeDtypeStruct((L, QCH), jnp.float32),
        ],
    )(xu_p, w1, b1, wq, bqr, wk, bkr)

    h1u = _unfold_perm(h1.T.reshape(CIN, H, W))
    quf = _unfold_perm(q.T.reshape(QCH, H, W))
    kuf = _unfold_perm(k.T.reshape(QCH, H, W))

    nb = L // LB_B
    feat, vals, idxs = pl.pallas_call(
        _stage_b_body,
        grid=(nb,),
        in_specs=[
            pl.BlockSpec((LB_B, 9 * CIN), lambda i: (i, 0)),
            pl.BlockSpec((9 * CIN, CIN), lambda i: (0, 0)),
            pl.BlockSpec((1, CIN), lambda i: (0, 0)),
            pl.BlockSpec((LB_B, 9 * QCH), lambda i: (i, 0)),
            pl.BlockSpec((L, 9 * QCH), lambda i: (0, 0)),
        ],
        out_specs=[
            pl.BlockSpec((LB_B, CIN), lambda i: (i, 0)),
            pl.BlockSpec((LB_B, 8), lambda i: (i, 0)),
            pl.BlockSpec((LB_B, 8), lambda i: (i, 0)),
        ],
        out_shape=[
            jax.ShapeDtypeStruct((L, CIN), jnp.float32),
            jax.ShapeDtypeStruct((L, 8), jnp.float32),
            jax.ShapeDtypeStruct((L, 8), jnp.int32),
        ],
    )(h1u, w2, b2, quf, kuf)

    idx_flat = idxs[:, 1:TOPK5].T.reshape(4 * L)
    g = _gather_rows(xu_p, idx_flat).reshape(4, L, GW)

    nd = L // LB_D
    yt = pl.pallas_call(
        _stage_d_body,
        grid=(nd,),
        in_specs=[
            pl.BlockSpec((4, LB_D, GW),
                         lambda i: (0, jnp.maximum(i - 1, 0), 0)),
            pl.BlockSpec((4, LB_D, GW), lambda i: (0, i, 0)),
            pl.BlockSpec((4, LB_D, GW),
                         lambda i: (0, jnp.minimum(i + 1, nd - 1), 0)),
            pl.BlockSpec((LB_D, 8), lambda i: (i, 0)),
            pl.BlockSpec((LB_D, CIN), lambda i: (i, 0)),
            pl.BlockSpec((LB_D, CIN), lambda i: (i, 0)),
        ],
        out_specs=pl.BlockSpec((LB_D, CIN), lambda i: (i, 0)),
        out_shape=jax.ShapeDtypeStruct((L, CIN), jnp.float32),
    )(g, g, g, vals, x.reshape(CIN, L).T, feat)

    return yt.T.reshape(1, CIN, H, W)


# speedup vs baseline: 20.1251x; 1.3549x over previous
import functools

import jax
import jax.numpy as jnp
from jax import lax
from jax.experimental import pallas as pl
from jax.experimental.pallas import tpu as pltpu
from jax.experimental.pallas import tpu_sc as plsc

H = 64
W = 64
L = H * W
CIN = 64
QCH = 16
TOPK5 = 5
LB_B = 512
GW = 640
MARG = 128
NEG = -3.0e38
OFFS = [(di, dj) for di in (-1, 0, 1) for dj in (-1, 0, 1)]


def _wflat(w):
    o, i = w.shape[0], w.shape[1]
    return w.transpose(2, 3, 1, 0).reshape(9 * i, o)


def _unfold_from_cat(cat, base, lb, yy, xx, c):
    cols = []
    for di, dj in OFFS:
        start = lb + 64 * di + dj
        sh = cat[start:start + lb, :]
        valid = ((yy + di) >= 0) & ((yy + di) < H) & \
                ((xx + dj) >= 0) & ((xx + dj) < W)
        cols.append(jnp.where(valid, sh, 0.0))
    return jnp.concatenate(cols, axis=1)


def _stage_a_body(x_ref, w1_ref, b1_ref, wq_ref, bq_ref, wk_ref, bk_ref,
                  h1_ref, q_ref, k_ref, xu_ref):
    x2 = x_ref[...]
    z = jnp.zeros((128, CIN), jnp.float32)
    cat = jnp.concatenate([z, x2, z], axis=0)
    yy = lax.broadcasted_iota(jnp.int32, (L, 1), 0) // W
    xx = lax.broadcasted_iota(jnp.int32, (L, 1), 0) % W
    cols = []
    for di, dj in OFFS:
        start = 128 + 64 * di + dj
        sh = cat[start:start + L, :]
        valid = ((yy + di) >= 0) & ((yy + di) < H) & \
                ((xx + dj) >= 0) & ((xx + dj) < W)
        cols.append(jnp.where(valid, sh, 0.0))
    xu = jnp.concatenate(cols, axis=1)
    xu_ref[...] = jnp.concatenate(
        [xu, jnp.zeros((L, GW - 9 * CIN), jnp.float32)], axis=1)
    h1 = jnp.dot(xu, w1_ref[...], preferred_element_type=jnp.float32)
    h1_ref[...] = jnp.maximum(h1 + b1_ref[...], 0.0)
    q_ref[...] = jnp.dot(xu, wq_ref[...],
                         preferred_element_type=jnp.float32) + bq_ref[...]
    k_ref[...] = jnp.dot(xu, wk_ref[...],
                         preferred_element_type=jnp.float32) + bk_ref[...]


def _stage_b_body(h1p_ref, h1c_ref, h1n_ref, qp_ref, qc_ref, qn_ref,
                  k_ref, w2_ref, b2_ref, feat_ref, vals_ref, idxs_ref):
    pid = pl.program_id(0)
    l_loc = lax.broadcasted_iota(jnp.int32, (LB_B, 1), 0) + pid * LB_B
    yy = l_loc // W
    xx = l_loc % W

    cat_h = jnp.concatenate([h1p_ref[...], h1c_ref[...], h1n_ref[...]],
                            axis=0)
    h1u = _unfold_from_cat(cat_h, None, LB_B, yy, xx, CIN)
    f = jnp.dot(h1u, w2_ref[...], preferred_element_type=jnp.float32)
    feat_ref[...] = jnp.maximum(f + b2_ref[...], 0.0)

    cat_q = jnp.concatenate([qp_ref[...], qc_ref[...], qn_ref[...]],
                            axis=0)
    qu = _unfold_from_cat(cat_q, None, LB_B, yy, xx, QCH)
    qn = qu / jnp.maximum(
        jnp.sqrt(jnp.sum(qu * qu, axis=1, keepdims=True)), 1e-12)

    kf = k_ref[...]
    zk = jnp.zeros((128, QCH), jnp.float32)
    cat_k = jnp.concatenate([zk, kf, zk], axis=0)
    yk = lax.broadcasted_iota(jnp.int32, (L, 1), 0) // W
    xk = lax.broadcasted_iota(jnp.int32, (L, 1), 0) % W
    kcols = []
    for di, dj in OFFS:
        start = 128 + 64 * di + dj
        sh = cat_k[start:start + L, :]
        valid = ((yk + di) >= 0) & ((yk + di) < H) & \
                ((xk + dj) >= 0) & ((xk + dj) < W)
        kcols.append(jnp.where(valid, sh, 0.0))
    ku = jnp.concatenate(kcols, axis=1)
    kn = ku / jnp.maximum(
        jnp.sqrt(jnp.sum(ku * ku, axis=1, keepdims=True)), 1e-12)

    r = lax.dot_general(qn, kn, (((1,), (1,)), ((), ())),
                        preferred_element_type=jnp.float32)
    iota = lax.broadcasted_iota(jnp.int32, (LB_B, L), 1)
    vs, ids = [], []
    for _ in range(TOPK5):
        m = jnp.max(r, axis=1, keepdims=True)
        idx = jnp.min(jnp.where(r == m, iota, L), axis=1, keepdims=True)
        vs.append(m)
        ids.append(idx)
        r = jnp.where(iota == idx, NEG, r)
    vals_ref[...] = jnp.concatenate(
        vs + [jnp.zeros((LB_B, 3), jnp.float32)], axis=1)
    idxs_ref[...] = jnp.concatenate(
        ids + [jnp.zeros((LB_B, 3), jnp.int32)], axis=1)


def _stage_d_body(g_ref, vals_ref, x_ref, feat_ref, out_ref):
    t = pl.program_id(0)
    yy = lax.broadcasted_iota(jnp.int32, (L, 1), 0) // W
    xx = lax.broadcasted_iota(jnp.int32, (L, 1), 0) % W
    lane = lax.broadcasted_iota(jnp.int32, (L, 8), 1)
    s = jnp.sum(jnp.where(lane == t + 1, vals_ref[...], 0.0),
                axis=1, keepdims=True)
    contrib = jnp.zeros((L, CIN), jnp.float32)
    for di, dj in OFFS:
        kap = (di + 1) * 3 + (dj + 1)
        start = MARG - 64 * di - dj
        sh = g_ref[0, start:start + L, kap * CIN:(kap + 1) * CIN]
        valid = ((yy - di) >= 0) & ((yy - di) < H) & \
                ((xx - dj) >= 0) & ((xx - dj) < W)
        contrib = contrib + jnp.where(valid, sh, 0.0)
    acc = s * contrib * (1.0 / 36.0)

    @pl.when(t == 0)
    def _():
        out_ref[...] = x_ref[...] + feat_ref[...] + acc

    @pl.when(t > 0)
    def _():
        out_ref[...] = out_ref[...] + acc


def _gather_rows(table, idx_flat):
    info = plsc.get_sparse_core_info()
    nw = info.num_cores * info.num_subcores
    n = idx_flat.shape[0]
    d = table.shape[1]
    per_w = n // nw
    ch = 128
    mesh = plsc.VectorSubcoreMesh(core_axis_name="c", subcore_axis_name="s")

    @functools.partial(
        pl.kernel, mesh=mesh,
        out_type=jax.ShapeDtypeStruct((4, L + 2 * MARG, d), jnp.float32),
        scratch_types=[
            pltpu.VMEM((ch,), jnp.int32),
            pltpu.VMEM((ch, d), jnp.float32),
            pltpu.SemaphoreType.DMA,
        ],
    )
    def k(table_hbm, idx_hbm, out_hbm, idx_v, rows_v, sem):
        wid = lax.axis_index("s") * info.num_cores + lax.axis_index("c")
        base = wid * per_w
        t = base // L
        lbase = base % L
        for c in range(per_w // ch):
            pltpu.sync_copy(idx_hbm.at[pl.ds(base + c * ch, ch)], idx_v)
            pltpu.async_copy(table_hbm.at[idx_v], rows_v, sem).wait()
            pltpu.sync_copy(
                rows_v, out_hbm.at[t, pl.ds(MARG + lbase + c * ch, ch)])

    return k(table, idx_flat)


def kernel(x, Wf1, bf1, Wf2, bf2, Wq, bq, Wk, bk):
    x2d = x.reshape(CIN, L).T
    w1 = _wflat(Wf1)
    w2 = _wflat(Wf2)
    wq = _wflat(Wq)
    wk = _wflat(Wk)
    b1 = bf1.reshape(1, CIN)
    b2 = bf2.reshape(1, CIN)
    bqr = bq.reshape(1, QCH)
    bkr = bk.reshape(1, QCH)

    h1, q, k, xu_p = pl.pallas_call(
        _stage_a_body,
        out_shape=[
            jax.ShapeDtypeStruct((L, CIN), jnp.float32),
            jax.ShapeDtypeStruct((L, QCH), jnp.float32),
            jax.ShapeDtypeStruct((L, QCH), jnp.float32),
            jax.ShapeDtypeStruct((L, GW), jnp.float32),
        ],
    )(x2d, w1, b1, wq, bqr, wk, bkr)

    nb = L // LB_B
    prev = lambda i: (jnp.maximum(i - 1, 0), 0)
    cur = lambda i: (i, 0)
    nxt = lambda i: (jnp.minimum(i + 1, nb - 1), 0)
    feat, vals, idxs = pl.pallas_call(
        _stage_b_body,
        grid=(nb,),
        in_specs=[
            pl.BlockSpec((LB_B, CIN), prev),
            pl.BlockSpec((LB_B, CIN), cur),
            pl.BlockSpec((LB_B, CIN), nxt),
            pl.BlockSpec((LB_B, QCH), prev),
            pl.BlockSpec((LB_B, QCH), cur),
            pl.BlockSpec((LB_B, QCH), nxt),
            pl.BlockSpec((L, QCH), lambda i: (0, 0)),
            pl.BlockSpec((9 * CIN, CIN), lambda i: (0, 0)),
            pl.BlockSpec((1, CIN), lambda i: (0, 0)),
        ],
        out_specs=[
            pl.BlockSpec((LB_B, CIN), cur),
            pl.BlockSpec((LB_B, 8), cur),
            pl.BlockSpec((LB_B, 8), cur),
        ],
        out_shape=[
            jax.ShapeDtypeStruct((L, CIN), jnp.float32),
            jax.ShapeDtypeStruct((L, 8), jnp.float32),
            jax.ShapeDtypeStruct((L, 8), jnp.int32),
        ],
    )(h1, h1, h1, q, q, q, k, w2, b2)

    idx_flat = idxs[:, 1:TOPK5].T.reshape(4 * L)
    g = _gather_rows(xu_p, idx_flat)

    yt = pl.pallas_call(
        _stage_d_body,
        grid=(4,),
        in_specs=[
            pl.BlockSpec((1, L + 2 * MARG, GW), lambda t: (t, 0, 0)),
            pl.BlockSpec((L, 8), lambda t: (0, 0)),
            pl.BlockSpec((L, CIN), lambda t: (0, 0)),
            pl.BlockSpec((L, CIN), lambda t: (0, 0)),
        ],
        out_specs=pl.BlockSpec((L, CIN), lambda t: (0, 0)),
        out_shape=jax.ShapeDtypeStruct((L, CIN), jnp.float32),
    )(g, vals, x2d, feat)

    return yt.T.reshape(1, CIN, H, W)


# speedup vs baseline: 26.8607x; 1.3347x over previous
import functools

import jax
import jax.numpy as jnp
from jax import lax
from jax.experimental import pallas as pl
from jax.experimental.pallas import tpu as pltpu
from jax.experimental.pallas import tpu_sc as plsc

H = 64
W = 64
L = H * W
CIN = 64
QCH = 16
TOPK5 = 5
LB_B = 512
GW = 640
MARG = 128
NEG = -3.0e38
OFFS = [(di, dj) for di in (-1, 0, 1) for dj in (-1, 0, 1)]


def _wflat(w):
    o, i = w.shape[0], w.shape[1]
    return w.transpose(2, 3, 1, 0).reshape(9 * i, o)


def _unfold_from_cat(cat, base, lb, yy, xx, c):
    cols = []
    for di, dj in OFFS:
        start = lb + 64 * di + dj
        sh = cat[start:start + lb, :]
        valid = ((yy + di) >= 0) & ((yy + di) < H) & \
                ((xx + dj) >= 0) & ((xx + dj) < W)
        cols.append(jnp.where(valid, sh, 0.0))
    return jnp.concatenate(cols, axis=1)


def _unfold_full(v, c, yy, xx):
    z = jnp.zeros((128, c), jnp.float32)
    cat = jnp.concatenate([z, v, z], axis=0)
    cols = []
    for di, dj in OFFS:
        start = 128 + 64 * di + dj
        sh = cat[start:start + L, :]
        valid = ((yy + di) >= 0) & ((yy + di) < H) & \
                ((xx + dj) >= 0) & ((xx + dj) < W)
        cols.append(jnp.where(valid, sh, 0.0))
    return jnp.concatenate(cols, axis=1)


def _stage_a_body(x_ref, wall_ref, ball_ref, w2_ref, b2_ref,
                  feat_ref, qn_ref, kn_ref, xu_ref):
    x2 = x_ref[...]
    yy = lax.broadcasted_iota(jnp.int32, (L, 1), 0) // W
    xx = lax.broadcasted_iota(jnp.int32, (L, 1), 0) % W
    xu = _unfold_full(x2, CIN, yy, xx)
    xu_ref[...] = jnp.concatenate(
        [xu, jnp.zeros((L, GW - 9 * CIN), jnp.float32)], axis=1)
    hqk = jnp.dot(xu, wall_ref[...],
                  preferred_element_type=jnp.float32) + ball_ref[...]
    h1 = jnp.maximum(hqk[:, :CIN], 0.0)
    q = hqk[:, CIN:CIN + QCH]
    k = hqk[:, CIN + QCH:CIN + 2 * QCH]
    h1u = _unfold_full(h1, CIN, yy, xx)
    f = jnp.dot(h1u, w2_ref[...], preferred_element_type=jnp.float32)
    feat_ref[...] = jnp.maximum(f + b2_ref[...], 0.0)
    qu = _unfold_full(q, QCH, yy, xx)
    qn_ref[...] = qu * lax.rsqrt(
        jnp.maximum(jnp.sum(qu * qu, axis=1, keepdims=True), 1e-24))
    ku = _unfold_full(k, QCH, yy, xx)
    kn_ref[...] = ku * lax.rsqrt(
        jnp.maximum(jnp.sum(ku * ku, axis=1, keepdims=True), 1e-24))


def _stage_b_body(qn_ref, kn_ref, vals_ref, idxs_ref):
    r = lax.dot_general(qn_ref[...], kn_ref[...], (((1,), (1,)), ((), ())),
                        preferred_element_type=jnp.float32)
    iota = lax.broadcasted_iota(jnp.int32, (LB_B, L), 1)
    vs, ids = [], []
    for _ in range(TOPK5):
        m = jnp.max(r, axis=1, keepdims=True)
        idx = jnp.min(jnp.where(r == m, iota, L), axis=1, keepdims=True)
        vs.append(m)
        ids.append(idx)
        r = jnp.where(iota == idx, NEG, r)
    vals_ref[...] = jnp.concatenate(
        vs + [jnp.zeros((LB_B, 3), jnp.float32)], axis=1)
    idxs_ref[...] = jnp.concatenate(
        ids + [jnp.zeros((LB_B, 3), jnp.int32)], axis=1)


def _stage_d_body(g_ref, vals_ref, x_ref, feat_ref, out_ref):
    t = pl.program_id(0)
    yy = lax.broadcasted_iota(jnp.int32, (L, 1), 0) // W
    xx = lax.broadcasted_iota(jnp.int32, (L, 1), 0) % W
    lane = lax.broadcasted_iota(jnp.int32, (L, 8), 1)
    s = jnp.sum(jnp.where(lane == t + 1, vals_ref[...], 0.0),
                axis=1, keepdims=True)
    contrib = jnp.zeros((L, CIN), jnp.float32)
    for di, dj in OFFS:
        kap = (di + 1) * 3 + (dj + 1)
        start = MARG - 64 * di - dj
        sh = g_ref[0, start:start + L, kap * CIN:(kap + 1) * CIN]
        valid = ((yy - di) >= 0) & ((yy - di) < H) & \
                ((xx - dj) >= 0) & ((xx - dj) < W)
        contrib = contrib + jnp.where(valid, sh, 0.0)
    acc = s * contrib * (1.0 / 36.0)

    @pl.when(t == 0)
    def _():
        out_ref[...] = x_ref[...] + feat_ref[...] + acc

    @pl.when(t > 0)
    def _():
        out_ref[...] = out_ref[...] + acc


def _gather_rows(table, idx_flat):
    info = plsc.get_sparse_core_info()
    nw = info.num_cores * info.num_subcores
    n = idx_flat.shape[0]
    d = table.shape[1]
    per_w = n // nw
    ch = 128
    mesh = plsc.VectorSubcoreMesh(core_axis_name="c", subcore_axis_name="s")

    @functools.partial(
        pl.kernel, mesh=mesh,
        out_type=jax.ShapeDtypeStruct((4, L + 2 * MARG, d), jnp.float32),
        scratch_types=[
            pltpu.VMEM((ch,), jnp.int32),
            pltpu.VMEM((ch, d), jnp.float32),
            pltpu.SemaphoreType.DMA,
        ],
    )
    def k(table_hbm, idx_hbm, out_hbm, idx_v, rows_v, sem):
        wid = lax.axis_index("s") * info.num_cores + lax.axis_index("c")
        base = wid * per_w
        t = base // L
        lbase = base % L
        for c in range(per_w // ch):
            pltpu.sync_copy(idx_hbm.at[pl.ds(base + c * ch, ch)], idx_v)
            pltpu.async_copy(table_hbm.at[idx_v], rows_v, sem).wait()
            pltpu.sync_copy(
                rows_v, out_hbm.at[t, pl.ds(MARG + lbase + c * ch, ch)])

    return k(table, idx_flat)


def kernel(x, Wf1, bf1, Wf2, bf2, Wq, bq, Wk, bk):
    x2d = x.reshape(CIN, L).T
    wall = jnp.concatenate([_wflat(Wf1), _wflat(Wq), _wflat(Wk)], axis=1)
    ball = jnp.concatenate([bf1, bq, bk]).reshape(1, CIN + 2 * QCH)
    w2 = _wflat(Wf2)
    b2 = bf2.reshape(1, CIN)

    feat, qn, kn, xu_p = pl.pallas_call(
        _stage_a_body,
        out_shape=[
            jax.ShapeDtypeStruct((L, CIN), jnp.float32),
            jax.ShapeDtypeStruct((L, 9 * QCH), jnp.float32),
            jax.ShapeDtypeStruct((L, 9 * QCH), jnp.float32),
            jax.ShapeDtypeStruct((L, GW), jnp.float32),
        ],
    )(x2d, wall, ball, w2, b2)

    nb = L // LB_B
    cur = lambda i: (i, 0)
    vals, idxs = pl.pallas_call(
        _stage_b_body,
        grid=(nb,),
        in_specs=[
            pl.BlockSpec((LB_B, 9 * QCH), cur),
            pl.BlockSpec((L, 9 * QCH), lambda i: (0, 0)),
        ],
        out_specs=[
            pl.BlockSpec((LB_B, 8), cur),
            pl.BlockSpec((LB_B, 8), cur),
        ],
        out_shape=[
            jax.ShapeDtypeStruct((L, 8), jnp.float32),
            jax.ShapeDtypeStruct((L, 8), jnp.int32),
        ],
    )(qn, kn)

    idx_flat = idxs[:, 1:TOPK5].T.reshape(4 * L)
    g = _gather_rows(xu_p, idx_flat)

    yt = pl.pallas_call(
        _stage_d_body,
        grid=(4,),
        in_specs=[
            pl.BlockSpec((1, L + 2 * MARG, GW), lambda t: (t, 0, 0)),
            pl.BlockSpec((L, 8), lambda t: (0, 0)),
            pl.BlockSpec((L, CIN), lambda t: (0, 0)),
            pl.BlockSpec((L, CIN), lambda t: (0, 0)),
        ],
        out_specs=pl.BlockSpec((L, CIN), lambda t: (0, 0)),
        out_shape=jax.ShapeDtypeStruct((L, CIN), jnp.float32),
    )(g, vals, x2d, feat)

    return yt.T.reshape(1, CIN, H, W)
